# Initial kernel scaffold; baseline (speedup 1.0000x reference)
import jax
import jax.numpy as jnp
from jax.experimental import pallas as pl


def kernel(x, table, W, U, b):
    raise NotImplementedError("write your pallas kernel here")


# speedup vs baseline: 2.9168x; 2.9168x over previous
import functools

import jax
import jax.numpy as jnp
from jax import lax
from jax.experimental import pallas as pl
from jax.experimental.pallas import tpu as pltpu
from jax.experimental.pallas import tpu_sc as plsc


_CHUNK = 64


def _sc_gather_body(table_hbm, idx_hbm, out_hbm, idx_v, rows_v, sem):
    nc = 2
    wid = lax.axis_index("s") * nc + lax.axis_index("c")
    n_chunks = idx_v.shape[0]
    rows_per_w = n_chunks * _CHUNK
    base = wid * rows_per_w
    pltpu.sync_copy(idx_hbm.at[wid], idx_v)

    def chunk(c, carry):
        pltpu.async_copy(table_hbm.at[idx_v.at[c]], rows_v, sem).wait()
        pltpu.sync_copy(rows_v, out_hbm.at[pl.ds(base + c * _CHUNK, _CHUNK)])
        return carry

    lax.fori_loop(0, n_chunks, chunk, 0)


def _sc_gather(table, idx_flat):
    n, e = idx_flat.shape[0], table.shape[1]
    info = plsc.get_sparse_core_info()
    nw = info.num_cores * info.num_subcores
    rows_per_w = n // nw
    n_chunks = rows_per_w // _CHUNK
    idx3 = idx_flat.reshape(nw, n_chunks, _CHUNK)
    mesh = plsc.VectorSubcoreMesh(core_axis_name="c", subcore_axis_name="s")
    return pl.kernel(
        _sc_gather_body,
        out_type=jax.ShapeDtypeStruct((n, e), jnp.float32),
        mesh=mesh,
        scratch_types=[
            pltpu.VMEM((n_chunks, _CHUNK), jnp.int32),
            pltpu.VMEM((_CHUNK, e), jnp.float32),
            pltpu.SemaphoreType.DMA,
        ],
    )(table, idx3)


def _gru_body(emb_ref, W_ref, U_ref, b_ref, out_ref, h_ref):
    t = pl.program_id(0)

    @pl.when(t == 0)
    def _init():
        h_ref[...] = jnp.zeros_like(h_ref)

    units = h_ref.shape[1]
    xt = emb_ref[0]
    h = h_ref[...]
    xw = jnp.dot(xt, W_ref[...], preferred_element_type=jnp.float32) + b_ref[0]
    hu = jnp.dot(h, U_ref[...], preferred_element_type=jnp.float32) + b_ref[1]
    xz = xw[:, :units]
    xr = xw[:, units:2 * units]
    xh = xw[:, 2 * units:]
    hz = hu[:, :units]
    hr = hu[:, units:2 * units]
    hh_lin = hu[:, 2 * units:]
    z = jax.nn.sigmoid(xz + hz)
    r = jax.nn.sigmoid(xr + hr)
    hh = jnp.tanh(xh + r * hh_lin)
    h_new = z * h + (1.0 - z) * hh
    h_ref[...] = h_new
    out_ref[0] = h_new


def _tc_gru(emb_tbe, W, U, b):
    t_len, batch, e = emb_tbe.shape
    units = U.shape[0]
    return pl.pallas_call(
        _gru_body,
        grid=(t_len,),
        in_specs=[
            pl.BlockSpec((1, batch, e), lambda t: (t, 0, 0)),
            pl.BlockSpec((e, 3 * units), lambda t: (0, 0)),
            pl.BlockSpec((units, 3 * units), lambda t: (0, 0)),
            pl.BlockSpec((2, 3 * units), lambda t: (0, 0)),
        ],
        out_specs=pl.BlockSpec((1, batch, units), lambda t: (t, 0, 0)),
        out_shape=jax.ShapeDtypeStruct((t_len, batch, units), jnp.float32),
        scratch_shapes=[pltpu.VMEM((batch, units), jnp.float32)],
    )(emb_tbe, W, U, b)


@jax.jit
def kernel(x, table, W, U, b):
    batch, t_len = x.shape
    e = table.shape[1]
    units = U.shape[0]
    idx_flat = jnp.swapaxes(x, 0, 1).reshape(-1)
    emb = _sc_gather(table, idx_flat)
    ys = _tc_gru(emb.reshape(t_len, batch, e), W, U, b)
    return jnp.swapaxes(ys, 0, 1)
